# baseline (speedup 1.0000x reference)
import jax
import jax.numpy as jnp
from jax.experimental import pallas as pl
from jax.experimental.pallas import tpu as pltpu


_S_BLK = 1024
_CHUNK = 512


def _coefs(base_row, d):
    r1b = pltpu.roll(base_row, d - 1, 1)
    r2b = pltpu.roll(base_row, 1, 1)
    evl = jax.lax.broadcasted_iota(jnp.int32, base_row.shape, 1) % 2 == 0
    coef_a = jnp.where(evl, r1b, base_row)
    coef_b = jnp.where(evl, base_row, -r2b)
    return coef_a, coef_b


def _pe_add_kernel(x_ref, offs_ref, basen_ref, o_ref, swap_ref, tab0_ref, tab1_ref):
    s = pl.program_id(0)
    b = pl.program_id(1)
    n_s = pl.num_programs(0)
    bs_steps = pl.num_programs(1)
    d = offs_ref.shape[-1]
    rows = offs_ref.shape[0]

    @pl.when(s < n_s - 1)
    def _build_next():
        coef_a, coef_b = _coefs(basen_ref[0:1, :], d)
        qrows = rows // bs_steps

        @pl.loop(0, qrows // _CHUNK)
        def _tab_chunk(i):
            sl = pl.ds(b * qrows + i * _CHUNK, _CHUNK)

            @pl.when(s == 0)
            def _swap_chunk():
                offs = offs_ref[sl, :]
                ev2 = (
                    jax.lax.broadcasted_iota(jnp.int32, offs.shape, 1) % 2 == 0
                )
                swap_ref[sl, :] = jnp.where(
                    ev2, pltpu.roll(offs, d - 1, 1), pltpu.roll(offs, 1, 1)
                )

            val = coef_a * offs_ref[sl, :] + coef_b * swap_ref[sl, :]

            @pl.when(s % 2 == 0)
            def _w1():
                tab1_ref[sl, :] = val

            @pl.when(s % 2 == 1)
            def _w0():
                tab0_ref[sl, :] = val

    @pl.when(s == 0)
    def _add_offs():
        o_ref[...] = x_ref[...] + offs_ref[...][None, :, :]

    @pl.when(jnp.logical_and(s > 0, s % 2 == 0))
    def _add0():
        o_ref[...] = x_ref[...] + tab0_ref[...][None, :, :]

    @pl.when(s % 2 == 1)
    def _add1():
        o_ref[...] = x_ref[...] + tab1_ref[...][None, :, :]


def kernel(x, pe):
    bs, seq, d = x.shape
    n_s = seq // _S_BLK
    bp = 2
    grid = (n_s, bs // bp)
    return pl.pallas_call(
        _pe_add_kernel,
        grid=grid,
        in_specs=[
            pl.BlockSpec((bp, _S_BLK, d), lambda s, b: (b, s, 0)),
            pl.BlockSpec((_S_BLK, d), lambda s, b: (0, 0)),
            pl.BlockSpec(
                (8, d),
                lambda s, b, n_s=n_s: (
                    jnp.minimum(s + 1, n_s - 1) * (_S_BLK // 8),
                    0,
                ),
            ),
        ],
        out_specs=pl.BlockSpec((bp, _S_BLK, d), lambda s, b: (b, s, 0)),
        out_shape=jax.ShapeDtypeStruct((bs, seq, d), x.dtype),
        scratch_shapes=[
            pltpu.VMEM((_S_BLK, d), jnp.float32),
            pltpu.VMEM((_S_BLK, d), jnp.float32),
            pltpu.VMEM((_S_BLK, d), jnp.float32),
        ],
    )(x, pe, pe)


# speedup vs baseline: 1.0026x; 1.0026x over previous
import jax
import jax.numpy as jnp
from jax.experimental import pallas as pl
from jax.experimental.pallas import tpu as pltpu


_S_BLK = 1024
_CHUNK = 256


def _coefs(base_row, d):
    r1b = pltpu.roll(base_row, d - 1, 1)
    r2b = pltpu.roll(base_row, 1, 1)
    evl = jax.lax.broadcasted_iota(jnp.int32, base_row.shape, 1) % 2 == 0
    coef_a = jnp.where(evl, r1b, base_row)
    coef_b = jnp.where(evl, base_row, -r2b)
    return coef_a, coef_b


def _pe_add_kernel(x_ref, offs_ref, basen_ref, o_ref, swap_ref, tab0_ref, tab1_ref):
    s = pl.program_id(0)
    b = pl.program_id(1)
    n_s = pl.num_programs(0)
    bs_steps = pl.num_programs(1)
    d = offs_ref.shape[-1]
    rows = offs_ref.shape[0]

    @pl.when(s < n_s - 1)
    def _build_next():
        coef_a, coef_b = _coefs(basen_ref[0:1, :], d)
        qrows = rows // bs_steps

        @pl.loop(0, qrows // _CHUNK)
        def _tab_chunk(i):
            sl = pl.ds(b * qrows + i * _CHUNK, _CHUNK)

            @pl.when(s == 0)
            def _swap_chunk():
                offs = offs_ref[sl, :]
                ev2 = (
                    jax.lax.broadcasted_iota(jnp.int32, offs.shape, 1) % 2 == 0
                )
                swap_ref[sl, :] = jnp.where(
                    ev2, pltpu.roll(offs, d - 1, 1), pltpu.roll(offs, 1, 1)
                )

            val = coef_a * offs_ref[sl, :] + coef_b * swap_ref[sl, :]

            @pl.when(s % 2 == 0)
            def _w1():
                tab1_ref[sl, :] = val

            @pl.when(s % 2 == 1)
            def _w0():
                tab0_ref[sl, :] = val

    @pl.when(s == 0)
    def _add_offs():
        o_ref[...] = x_ref[...] + offs_ref[...][None, :, :]

    @pl.when(jnp.logical_and(s > 0, s % 2 == 0))
    def _add0():
        o_ref[...] = x_ref[...] + tab0_ref[...][None, :, :]

    @pl.when(s % 2 == 1)
    def _add1():
        o_ref[...] = x_ref[...] + tab1_ref[...][None, :, :]


def kernel(x, pe):
    bs, seq, d = x.shape
    n_s = seq // _S_BLK
    bp = 2
    grid = (n_s, bs // bp)
    return pl.pallas_call(
        _pe_add_kernel,
        grid=grid,
        in_specs=[
            pl.BlockSpec((bp, _S_BLK, d), lambda s, b: (b, s, 0)),
            pl.BlockSpec((_S_BLK, d), lambda s, b: (0, 0)),
            pl.BlockSpec(
                (8, d),
                lambda s, b, n_s=n_s: (
                    jnp.minimum(s + 1, n_s - 1) * (_S_BLK // 8),
                    0,
                ),
            ),
        ],
        out_specs=pl.BlockSpec((bp, _S_BLK, d), lambda s, b: (b, s, 0)),
        out_shape=jax.ShapeDtypeStruct((bs, seq, d), x.dtype),
        scratch_shapes=[
            pltpu.VMEM((_S_BLK, d), jnp.float32),
            pltpu.VMEM((_S_BLK, d), jnp.float32),
            pltpu.VMEM((_S_BLK, d), jnp.float32),
        ],
    )(x, pe, pe)


# speedup vs baseline: 1.0084x; 1.0058x over previous
import jax
import jax.numpy as jnp
from jax.experimental import pallas as pl
from jax.experimental.pallas import tpu as pltpu


_S_BLK = 512
_CHUNK = 256


def _coefs(base_row, d):
    r1b = pltpu.roll(base_row, d - 1, 1)
    r2b = pltpu.roll(base_row, 1, 1)
    evl = jax.lax.broadcasted_iota(jnp.int32, base_row.shape, 1) % 2 == 0
    coef_a = jnp.where(evl, r1b, base_row)
    coef_b = jnp.where(evl, base_row, -r2b)
    return coef_a, coef_b


def _pe_add_kernel(x_ref, offs_ref, basen_ref, o_ref, swap_ref, tab0_ref, tab1_ref):
    s = pl.program_id(0)
    b = pl.program_id(1)
    n_s = pl.num_programs(0)
    bs_steps = pl.num_programs(1)
    d = offs_ref.shape[-1]
    rows = offs_ref.shape[0]

    @pl.when(s < n_s - 1)
    def _build_next():
        coef_a, coef_b = _coefs(basen_ref[0:1, :], d)
        qrows = rows // bs_steps

        @pl.loop(0, qrows // _CHUNK)
        def _tab_chunk(i):
            sl = pl.ds(b * qrows + i * _CHUNK, _CHUNK)

            @pl.when(s == 0)
            def _swap_chunk():
                offs = offs_ref[sl, :]
                ev2 = (
                    jax.lax.broadcasted_iota(jnp.int32, offs.shape, 1) % 2 == 0
                )
                swap_ref[sl, :] = jnp.where(
                    ev2, pltpu.roll(offs, d - 1, 1), pltpu.roll(offs, 1, 1)
                )

            val = coef_a * offs_ref[sl, :] + coef_b * swap_ref[sl, :]

            @pl.when(s % 2 == 0)
            def _w1():
                tab1_ref[sl, :] = val

            @pl.when(s % 2 == 1)
            def _w0():
                tab0_ref[sl, :] = val

    @pl.when(s == 0)
    def _add_offs():
        o_ref[...] = x_ref[...] + offs_ref[...][None, :, :]

    @pl.when(jnp.logical_and(s > 0, s % 2 == 0))
    def _add0():
        o_ref[...] = x_ref[...] + tab0_ref[...][None, :, :]

    @pl.when(s % 2 == 1)
    def _add1():
        o_ref[...] = x_ref[...] + tab1_ref[...][None, :, :]


def kernel(x, pe):
    bs, seq, d = x.shape
    n_s = seq // _S_BLK
    bp = 4
    grid = (n_s, bs // bp)
    return pl.pallas_call(
        _pe_add_kernel,
        grid=grid,
        in_specs=[
            pl.BlockSpec((bp, _S_BLK, d), lambda s, b: (b, s, 0)),
            pl.BlockSpec((_S_BLK, d), lambda s, b: (0, 0)),
            pl.BlockSpec(
                (8, d),
                lambda s, b, n_s=n_s: (
                    jnp.minimum(s + 1, n_s - 1) * (_S_BLK // 8),
                    0,
                ),
            ),
        ],
        out_specs=pl.BlockSpec((bp, _S_BLK, d), lambda s, b: (b, s, 0)),
        out_shape=jax.ShapeDtypeStruct((bs, seq, d), x.dtype),
        scratch_shapes=[
            pltpu.VMEM((_S_BLK, d), jnp.float32),
            pltpu.VMEM((_S_BLK, d), jnp.float32),
            pltpu.VMEM((_S_BLK, d), jnp.float32),
        ],
    )(x, pe, pe)


# speedup vs baseline: 1.0088x; 1.0003x over previous
import jax
import jax.numpy as jnp
from jax.experimental import pallas as pl
from jax.experimental.pallas import tpu as pltpu


_S_BLK = 512
_CHUNK = 512


def _coefs(base_row, d):
    r1b = pltpu.roll(base_row, d - 1, 1)
    r2b = pltpu.roll(base_row, 1, 1)
    evl = jax.lax.broadcasted_iota(jnp.int32, base_row.shape, 1) % 2 == 0
    coef_a = jnp.where(evl, r1b, base_row)
    coef_b = jnp.where(evl, base_row, -r2b)
    return coef_a, coef_b


def _pe_add_kernel(x_ref, offs_ref, basen_ref, o_ref, swap_ref, tab0_ref, tab1_ref):
    s = pl.program_id(0)
    b = pl.program_id(1)
    n_s = pl.num_programs(0)
    bs_steps = pl.num_programs(1)
    d = offs_ref.shape[-1]
    rows = offs_ref.shape[0]

    @pl.when(s < n_s - 1)
    def _build_next():
        coef_a, coef_b = _coefs(basen_ref[0:1, :], d)
        qrows = rows // bs_steps

        @pl.loop(0, qrows // _CHUNK)
        def _tab_chunk(i):
            sl = pl.ds(b * qrows + i * _CHUNK, _CHUNK)

            @pl.when(s == 0)
            def _swap_chunk():
                offs = offs_ref[sl, :]
                ev2 = (
                    jax.lax.broadcasted_iota(jnp.int32, offs.shape, 1) % 2 == 0
                )
                swap_ref[sl, :] = jnp.where(
                    ev2, pltpu.roll(offs, d - 1, 1), pltpu.roll(offs, 1, 1)
                )

            val = coef_a * offs_ref[sl, :] + coef_b * swap_ref[sl, :]

            @pl.when(s % 2 == 0)
            def _w1():
                tab1_ref[sl, :] = val

            @pl.when(s % 2 == 1)
            def _w0():
                tab0_ref[sl, :] = val

    @pl.when(s == 0)
    def _add_offs():
        o_ref[...] = x_ref[...] + offs_ref[...][None, :, :]

    @pl.when(jnp.logical_and(s > 0, s % 2 == 0))
    def _add0():
        o_ref[...] = x_ref[...] + tab0_ref[...][None, :, :]

    @pl.when(s % 2 == 1)
    def _add1():
        o_ref[...] = x_ref[...] + tab1_ref[...][None, :, :]


def kernel(x, pe):
    bs, seq, d = x.shape
    n_s = seq // _S_BLK
    bp = 4
    grid = (n_s, bs // bp)
    return pl.pallas_call(
        _pe_add_kernel,
        grid=grid,
        in_specs=[
            pl.BlockSpec((bp, _S_BLK, d), lambda s, b: (b, s, 0)),
            pl.BlockSpec((_S_BLK, d), lambda s, b: (0, 0)),
            pl.BlockSpec(
                (8, d),
                lambda s, b, n_s=n_s: (
                    jnp.minimum(s + 1, n_s - 1) * (_S_BLK // 8),
                    0,
                ),
            ),
        ],
        out_specs=pl.BlockSpec((bp, _S_BLK, d), lambda s, b: (b, s, 0)),
        out_shape=jax.ShapeDtypeStruct((bs, seq, d), x.dtype),
        scratch_shapes=[
            pltpu.VMEM((_S_BLK, d), jnp.float32),
            pltpu.VMEM((_S_BLK, d), jnp.float32),
            pltpu.VMEM((_S_BLK, d), jnp.float32),
        ],
    )(x, pe, pe)
